# Initial kernel scaffold; baseline (speedup 1.0000x reference)
import jax
import jax.numpy as jnp
from jax.experimental import pallas as pl


def kernel(pos, edge_index, atom_types, mol_ids, vdw_radii):
    raise NotImplementedError("write your pallas kernel here")


# speedup vs baseline: 507.7285x; 507.7285x over previous
import functools

import jax
import jax.numpy as jnp
from jax import lax
from jax.experimental import pallas as pl
from jax.experimental.pallas import tpu as pltpu
from jax.experimental.pallas import tpu_sc as plsc

N_NODES = 10000
N_EDGES = 640000
N_TYPES_PAD = 112
NUM_GRAPHS = 256
TURNOVER = 10.0

NC = 2
NS = 16
NW = NC * NS
L = 16

EDGES_PER_W = N_EDGES // NW
EDGE_ITERS = EDGES_PER_W // L
NODE_ITERS = N_NODES // L
ACC_FLAT = L * NUM_GRAPHS


def _rsqrt16(x):
    xi = plsc.bitcast(x, jnp.int32)
    yi = jnp.int32(0x5F3759DF) - lax.shift_right_logical(xi, jnp.int32(1))
    y = plsc.bitcast(yi, jnp.float32)
    half = x * 0.5
    for _ in range(3):
        y = y * (1.5 - half * y * y)
    return y


def _sc_body(src_hbm, dst_hbm, px_hbm, py_hbm, pz_hbm, at_hbm, mi_hbm,
             rad_hbm, out_lj_hbm, out_no_hbm,
             src_v, dst_v, px_v, py_v, pz_v, at_v, mi_v, rad_v, rn_v,
             acc_lj, acc_no):
    wid = lax.axis_index("s") * NC + lax.axis_index("c")
    base = wid * EDGES_PER_W

    pltpu.sync_copy(src_hbm.at[pl.ds(base, EDGES_PER_W)], src_v)
    pltpu.sync_copy(dst_hbm.at[pl.ds(base, EDGES_PER_W)], dst_v)
    pltpu.sync_copy(px_hbm, px_v)
    pltpu.sync_copy(py_hbm, py_v)
    pltpu.sync_copy(pz_hbm, pz_v)
    pltpu.sync_copy(at_hbm, at_v)
    pltpu.sync_copy(mi_hbm, mi_v)
    pltpu.sync_copy(rad_hbm, rad_v)

    zeros = jnp.zeros((L,), jnp.float32)
    l16 = jnp.int32(L)

    def _zero(k, _):
        off = k * l16
        acc_lj[pl.ds(off, L)] = zeros
        acc_no[pl.ds(off, L)] = zeros
        return 0

    lax.fori_loop(jnp.int32(0), jnp.int32(NUM_GRAPHS), _zero, 0)

    def _rn(j, _):
        off = j * l16
        at16 = at_v[pl.ds(off, L)]
        rn_v[pl.ds(off, L)] = plsc.load_gather(rad_v, [at16])
        return 0

    lax.fori_loop(jnp.int32(0), jnp.int32(NODE_ITERS), _rn, 0)

    lane = lax.iota(jnp.int32, L) * jnp.int32(NUM_GRAPHS)

    def _edge(i, _):
        off = i * l16
        s = src_v[pl.ds(off, L)]
        t = dst_v[pl.ds(off, L)]
        dx = plsc.load_gather(px_v, [s]) - plsc.load_gather(px_v, [t])
        dy = plsc.load_gather(py_v, [s]) - plsc.load_gather(py_v, [t])
        dz = plsc.load_gather(pz_v, [s]) - plsc.load_gather(pz_v, [t])
        rv = plsc.load_gather(rn_v, [s]) + plsc.load_gather(rn_v, [t])
        g = plsc.load_gather(mi_v, [t])
        ss = dx * dx + dy * dy + dz * dz + 1e-12
        d2 = jnp.maximum(ss, 0.0025)
        r2 = (rv * rv) / d2
        r6 = r2 * r2 * r2
        lj = r6 * r6 - 2.0 * r6
        d = d2 * _rsqrt16(d2)
        ov = jnp.maximum(rv - d, 0.0)
        no = ov / rv
        idx = lane + g
        plsc.addupdate_scatter(acc_lj, [idx], lj)
        plsc.addupdate_scatter(acc_no, [idx], no)
        return 0

    lax.fori_loop(jnp.int32(0), jnp.int32(EDGE_ITERS), _edge, 0)

    pltpu.sync_copy(acc_lj, out_lj_hbm.at[wid])
    pltpu.sync_copy(acc_no, out_no_hbm.at[wid])


def _tc_body(lj_ref, no_ref, pot_ref, loss_ref, nov_ref):
    pot = jnp.sum(lj_ref[...], axis=0, keepdims=True)
    nov = jnp.sum(no_ref[...], axis=0, keepdims=True)
    safe = jnp.maximum(pot, TURNOVER)
    loss = jnp.where(pot > TURNOVER,
                     TURNOVER * (1.0 + jnp.log(safe / TURNOVER)),
                     pot)
    pot_ref[...] = pot
    loss_ref[...] = loss
    nov_ref[...] = nov


def kernel(pos, edge_index, atom_types, mol_ids, vdw_radii):
    src = edge_index[0].astype(jnp.int32)
    dst = edge_index[1].astype(jnp.int32)
    pos32 = pos.astype(jnp.float32)
    px = pos32[:, 0]
    py = pos32[:, 1]
    pz = pos32[:, 2]
    at = atom_types.astype(jnp.int32)
    mi = mol_ids.astype(jnp.int32)
    rad = jnp.zeros((N_TYPES_PAD,), jnp.float32).at[: vdw_radii.shape[0]].set(
        vdw_radii.astype(jnp.float32))

    mesh = plsc.VectorSubcoreMesh(
        core_axis_name="c", subcore_axis_name="s",
        num_cores=NC, num_subcores=NS)

    sc = functools.partial(
        pl.kernel,
        out_type=[
            jax.ShapeDtypeStruct((NW, ACC_FLAT), jnp.float32),
            jax.ShapeDtypeStruct((NW, ACC_FLAT), jnp.float32),
        ],
        mesh=mesh,
        compiler_params=pltpu.CompilerParams(needs_layout_passes=False),
        scratch_types=[
            pltpu.VMEM((EDGES_PER_W,), jnp.int32),
            pltpu.VMEM((EDGES_PER_W,), jnp.int32),
            pltpu.VMEM((N_NODES,), jnp.float32),
            pltpu.VMEM((N_NODES,), jnp.float32),
            pltpu.VMEM((N_NODES,), jnp.float32),
            pltpu.VMEM((N_NODES,), jnp.int32),
            pltpu.VMEM((N_NODES,), jnp.int32),
            pltpu.VMEM((N_TYPES_PAD,), jnp.float32),
            pltpu.VMEM((N_NODES,), jnp.float32),
            pltpu.VMEM((ACC_FLAT,), jnp.float32),
            pltpu.VMEM((ACC_FLAT,), jnp.float32),
        ],
    )(_sc_body)

    part_lj, part_no = sc(src, dst, px, py, pz, at, mi, rad)

    pot, loss, nov = pl.pallas_call(
        _tc_body,
        out_shape=[jax.ShapeDtypeStruct((1, NUM_GRAPHS), jnp.float32)] * 3,
    )(part_lj.reshape(NW * L, NUM_GRAPHS), part_no.reshape(NW * L, NUM_GRAPHS))

    return (pot.reshape(NUM_GRAPHS), loss.reshape(NUM_GRAPHS),
            nov.reshape(NUM_GRAPHS))


# speedup vs baseline: 842.3448x; 1.6590x over previous
import functools

import jax
import jax.numpy as jnp
from jax import lax
from jax.experimental import pallas as pl
from jax.experimental.pallas import tpu as pltpu
from jax.experimental.pallas import tpu_sc as plsc

N_NODES = 10000
N_EDGES = 640000
N_TYPES_PAD = 112
NUM_GRAPHS = 256
TURNOVER = 10.0

NC = 2
NS = 16
NW = NC * NS
L = 16

EDGES_PER_W = N_EDGES // NW
EDGE_ITERS = EDGES_PER_W // L
NODE_ITERS = N_NODES // L
ACC_FLAT = L * NUM_GRAPHS


def _rsqrt16(x):
    xi = plsc.bitcast(x, jnp.int32)
    yi = jnp.int32(0x5F3759DF) - lax.shift_right_logical(xi, jnp.int32(1))
    y = plsc.bitcast(yi, jnp.float32)
    half = x * 0.5
    for _ in range(3):
        y = y * (1.5 - half * y * y)
    return y


def _sc_body(src_hbm, dst_hbm, px_hbm, py_hbm, pz_hbm, at_hbm, mi_hbm,
             rad_hbm, out_lj_hbm, out_no_hbm,
             src_v, dst_v, px_v, py_v, pz_v, at_v, mi_v, rad_v, rn_v,
             acc_lj, acc_no):
    wid = lax.axis_index("s") * NC + lax.axis_index("c")
    base = wid * EDGES_PER_W

    pltpu.sync_copy(src_hbm.at[pl.ds(base, EDGES_PER_W)], src_v)
    pltpu.sync_copy(dst_hbm.at[pl.ds(base, EDGES_PER_W)], dst_v)
    pltpu.sync_copy(px_hbm, px_v)
    pltpu.sync_copy(py_hbm, py_v)
    pltpu.sync_copy(pz_hbm, pz_v)
    pltpu.sync_copy(at_hbm, at_v)
    pltpu.sync_copy(mi_hbm, mi_v)
    pltpu.sync_copy(rad_hbm, rad_v)

    zeros = jnp.zeros((L,), jnp.float32)
    l16 = jnp.int32(L)

    @plsc.parallel_loop(jnp.int32(0), jnp.int32(ACC_FLAT), step=jnp.int32(L), unroll=8)
    def _zero(off):
        acc_lj[pl.ds(off, L)] = zeros
        acc_no[pl.ds(off, L)] = zeros

    @plsc.parallel_loop(jnp.int32(0), jnp.int32(N_NODES), step=jnp.int32(L), unroll=4)
    def _rn(off):
        at16 = at_v[pl.ds(off, L)]
        rn_v[pl.ds(off, L)] = plsc.load_gather(rad_v, [at16])

    lane = lax.iota(jnp.int32, L) * jnp.int32(NUM_GRAPHS)

    @plsc.parallel_loop(jnp.int32(0), jnp.int32(EDGES_PER_W), step=jnp.int32(L), unroll=4)
    def _edge(off):
        s = src_v[pl.ds(off, L)]
        t = dst_v[pl.ds(off, L)]
        dx = plsc.load_gather(px_v, [s]) - plsc.load_gather(px_v, [t])
        dy = plsc.load_gather(py_v, [s]) - plsc.load_gather(py_v, [t])
        dz = plsc.load_gather(pz_v, [s]) - plsc.load_gather(pz_v, [t])
        rv = plsc.load_gather(rn_v, [s]) + plsc.load_gather(rn_v, [t])
        g = plsc.load_gather(mi_v, [t])
        ss = dx * dx + dy * dy + dz * dz + 1e-12
        d2 = jnp.maximum(ss, 0.0025)
        r2 = (rv * rv) / d2
        r6 = r2 * r2 * r2
        lj = r6 * r6 - 2.0 * r6
        d = d2 * _rsqrt16(d2)
        ov = jnp.maximum(rv - d, 0.0)
        no = ov / rv
        idx = lane + g
        plsc.addupdate_scatter(acc_lj, [idx], lj)
        plsc.addupdate_scatter(acc_no, [idx], no)

    pltpu.sync_copy(acc_lj, out_lj_hbm.at[wid])
    pltpu.sync_copy(acc_no, out_no_hbm.at[wid])


def _tc_body(lj_ref, no_ref, pot_ref, loss_ref, nov_ref):
    pot = jnp.sum(lj_ref[...], axis=0, keepdims=True)
    nov = jnp.sum(no_ref[...], axis=0, keepdims=True)
    safe = jnp.maximum(pot, TURNOVER)
    loss = jnp.where(pot > TURNOVER,
                     TURNOVER * (1.0 + jnp.log(safe / TURNOVER)),
                     pot)
    pot_ref[...] = pot
    loss_ref[...] = loss
    nov_ref[...] = nov


def kernel(pos, edge_index, atom_types, mol_ids, vdw_radii):
    src = edge_index[0].astype(jnp.int32)
    dst = edge_index[1].astype(jnp.int32)
    pos32 = pos.astype(jnp.float32)
    px = pos32[:, 0]
    py = pos32[:, 1]
    pz = pos32[:, 2]
    at = atom_types.astype(jnp.int32)
    mi = mol_ids.astype(jnp.int32)
    rad = jnp.zeros((N_TYPES_PAD,), jnp.float32).at[: vdw_radii.shape[0]].set(
        vdw_radii.astype(jnp.float32))

    mesh = plsc.VectorSubcoreMesh(
        core_axis_name="c", subcore_axis_name="s",
        num_cores=NC, num_subcores=NS)

    sc = functools.partial(
        pl.kernel,
        out_type=[
            jax.ShapeDtypeStruct((NW, ACC_FLAT), jnp.float32),
            jax.ShapeDtypeStruct((NW, ACC_FLAT), jnp.float32),
        ],
        mesh=mesh,
        compiler_params=pltpu.CompilerParams(needs_layout_passes=False),
        scratch_types=[
            pltpu.VMEM((EDGES_PER_W,), jnp.int32),
            pltpu.VMEM((EDGES_PER_W,), jnp.int32),
            pltpu.VMEM((N_NODES,), jnp.float32),
            pltpu.VMEM((N_NODES,), jnp.float32),
            pltpu.VMEM((N_NODES,), jnp.float32),
            pltpu.VMEM((N_NODES,), jnp.int32),
            pltpu.VMEM((N_NODES,), jnp.int32),
            pltpu.VMEM((N_TYPES_PAD,), jnp.float32),
            pltpu.VMEM((N_NODES,), jnp.float32),
            pltpu.VMEM((ACC_FLAT,), jnp.float32),
            pltpu.VMEM((ACC_FLAT,), jnp.float32),
        ],
    )(_sc_body)

    part_lj, part_no = sc(src, dst, px, py, pz, at, mi, rad)

    pot, loss, nov = pl.pallas_call(
        _tc_body,
        out_shape=[jax.ShapeDtypeStruct((1, NUM_GRAPHS), jnp.float32)] * 3,
    )(part_lj.reshape(NW * L, NUM_GRAPHS), part_no.reshape(NW * L, NUM_GRAPHS))

    return (pot.reshape(NUM_GRAPHS), loss.reshape(NUM_GRAPHS),
            nov.reshape(NUM_GRAPHS))
